# baseline (speedup 1.0000x reference)
import functools

import jax
import jax.numpy as jnp
import numpy as np
from jax import lax
from jax.experimental import layout as jax_layout
from jax.experimental import pallas as pl
from jax.experimental.pallas import tpu as pltpu
from jax.experimental.pallas import tpu_sc as plsc

B = 4096
D = 128
C = D * (D + 1) // 2
L = 16
NC, NS = 2, 16
NW = NC * NS
RPW = B // NW
SR = 8
NSTR = RPW // SR
SC_END = 8064
HALF = 4096
HB = SC_END - HALF
PADL = 16
PADW = 176

_MUL_SKEW = 5
_ST_SKEW = 6


def _block_list():
    off = [0]
    for i in range(D):
        off.append(off[-1] + D - i)
    out = []
    for m in range(SC_END // L):
        c0 = L * m
        i = 0
        while off[i + 1] <= c0:
            i += 1
        j = int(c0 >= HALF)
        local = c0 - j * HALF
        if off[i + 1] >= c0 + L:
            out.append(("s", j, local, i, i + c0 - off[i]))
        else:
            d = off[i + 1] - c0
            la = i + c0 - off[i]
            lb = (i + 1) + c0 - off[i + 1]
            out.append(("b", j, local, i, la, i + 1, lb, d))
    return out


_BLOCKS = _block_list()


def _emit_row(x_s, q, ba, bb, masks):
    vs = [x_s[q, pl.ds(PADL + L * m, L)] for m in range(D // L)]
    splats = {}

    def splat(i):
        if i not in splats:
            splats[i] = vs[i // L][i % L]
        return splats[i]

    bufs = (ba, bb)
    n = len(_BLOCKS)
    loads = [None] * n
    prods = [None] * n
    for t in range(n + _ST_SKEW):
        if t < n:
            blk = _BLOCKS[t]
            if blk[0] == "s":
                loads[t] = (x_s[q, pl.ds(PADL + blk[4], L)],)
            else:
                loads[t] = (
                    x_s[q, pl.ds(PADL + blk[4], L)],
                    x_s[q, pl.ds(PADL + blk[6], L)],
                )
        tm = t - _MUL_SKEW
        if 0 <= tm < n:
            blk = _BLOCKS[tm]
            if blk[0] == "s":
                prods[tm] = splat(blk[3]) * loads[tm][0]
            else:
                va = splat(blk[3]) * loads[tm][0]
                vb = splat(blk[5]) * loads[tm][1]
                prods[tm] = jnp.where(masks[blk[7]], va, vb)
        ts = t - _ST_SKEW
        if 0 <= ts < n:
            blk = _BLOCKS[ts]
            bufs[blk[1]][q, pl.ds(blk[2], L)] = prods[ts]


def _sc_body(x_hbm, out_hbm, x_s, a0, b0, a1, b1, sa0, sb0, sa1, sb1):
    cid = lax.axis_index("c")
    sid = lax.axis_index("s")
    wid = sid * NC + cid
    base = wid * RPW
    sets = ((a0, b0, sa0, sb0), (a1, b1, sa1, sb1))
    iota = lax.broadcasted_iota(jnp.int32, (L,), 0)
    masks = {d: iota < d for d in range(1, L)}

    def stripe(t, s, ba, bb, sema, semb):
        row0 = base + s * SR
        pltpu.sync_copy(x_hbm.at[pl.ds(row0, SR)], x_s)

        @pl.when(t > 0)
        def _wait():
            pltpu.make_async_copy(
                ba, out_hbm.at[pl.ds(base, SR), pl.ds(0, HALF)], sema
            ).wait()
            pltpu.make_async_copy(
                bb, out_hbm.at[pl.ds(base, SR), pl.ds(HALF, HB)], semb
            ).wait()

        def row(q, carry):
            _emit_row(x_s, q, ba, bb, masks)
            return carry

        lax.fori_loop(0, SR, row, 0)
        pltpu.async_copy(ba, out_hbm.at[pl.ds(row0, SR), pl.ds(0, HALF)], sema)
        pltpu.async_copy(bb, out_hbm.at[pl.ds(row0, SR), pl.ds(HALF, HB)], semb)

    def pair(t, carry):
        for k, (ba, bb, sema, semb) in enumerate(sets):
            stripe(t, t * 2 + k, ba, bb, sema, semb)
        return carry

    lax.fori_loop(0, NSTR // 2, pair, 0)
    for ba, bb, sema, semb in sets:
        pltpu.make_async_copy(
            ba, out_hbm.at[pl.ds(base, SR), pl.ds(0, HALF)], sema
        ).wait()
        pltpu.make_async_copy(
            bb, out_hbm.at[pl.ds(base, SR), pl.ds(HALF, HB)], semb
        ).wait()


_mesh = plsc.VectorSubcoreMesh(core_axis_name="c", subcore_axis_name="s")

_sc_call = pl.kernel(
    _sc_body,
    out_type=jax.ShapeDtypeStruct((B, C), jnp.float32),
    mesh=_mesh,
    scratch_types=[
        pltpu.VMEM((SR, PADW), jnp.float32),
        pltpu.VMEM((SR, HALF), jnp.float32),
        pltpu.VMEM((SR, HB), jnp.float32),
        pltpu.VMEM((SR, HALF), jnp.float32),
        pltpu.VMEM((SR, HB), jnp.float32),
        pltpu.SemaphoreType.DMA,
        pltpu.SemaphoreType.DMA,
        pltpu.SemaphoreType.DMA,
        pltpu.SemaphoreType.DMA,
    ],
)


def _sel_matrices():
    pairs = []
    off = 0
    for i in range(D):
        for j in range(i, D):
            pairs.append((i, j))
    si = np.zeros((D, 256), np.float32)
    sj = np.zeros((D, 256), np.float32)
    for cc in range(256):
        c = SC_END + cc
        if c < C:
            i, j = pairs[c]
            si[i, cc] = 1.0
            sj[j, cc] = 1.0
    return si, sj


_SEL_I, _SEL_J = _sel_matrices()


def _tc_body(x_ref, si_ref, sj_ref, prev_ref, o_ref):
    xv = x_ref[...]
    a = jnp.dot(xv, si_ref[...], preferred_element_type=jnp.float32,
                precision=lax.Precision.HIGHEST)
    b = jnp.dot(xv, sj_ref[...], preferred_element_type=jnp.float32,
                precision=lax.Precision.HIGHEST)
    o_ref[...] = a * b


_tc_call = pl.pallas_call(
    _tc_body,
    grid=(8, 2),
    in_specs=[
        pl.BlockSpec((512, D), lambda b, j: (b, 0)),
        pl.BlockSpec((D, 128), lambda b, j: (0, j)),
        pl.BlockSpec((D, 128), lambda b, j: (0, j)),
        pl.BlockSpec((512, 128), lambda b, j: (b, 63 + j)),
    ],
    out_specs=pl.BlockSpec((512, 128), lambda b, j: (b, 63 + j)),
    out_shape=jax.ShapeDtypeStruct((B, C), jnp.float32),
    input_output_aliases={3: 0},
)


@jax.jit
def kernel(x):
    assert x.shape == (B, D) and x.dtype == jnp.float32
    xp = jnp.pad(x, ((0, 0), (PADL, PADW - PADL - D)))
    part = _sc_call(xp)
    out = _tc_call(x, _SEL_I, _SEL_J, part)
    return jax_layout.with_layout_constraint(out, jax_layout.Layout((0, 1)))


# speedup vs baseline: 1.0007x; 1.0007x over previous
import functools

import jax
import jax.numpy as jnp
import numpy as np
from jax import lax
from jax.experimental import pallas as pl
from jax.experimental.pallas import tpu as pltpu
from jax.experimental.pallas import tpu_sc as plsc

B = 4096
D = 128
C = D * (D + 1) // 2
L = 16
NC, NS = 2, 16
NW = NC * NS
RPW = B // NW
SR = 8
NSTR = RPW // SR
SC_END = 8064
HALF = 4096
HB = SC_END - HALF
PADL = 16
PADW = 176

_MUL_SKEW = 5
_ST_SKEW = 6


def _block_list():
    off = [0]
    for i in range(D):
        off.append(off[-1] + D - i)
    out = []
    for m in range(SC_END // L):
        c0 = L * m
        i = 0
        while off[i + 1] <= c0:
            i += 1
        j = int(c0 >= HALF)
        local = c0 - j * HALF
        if off[i + 1] >= c0 + L:
            out.append(("s", j, local, i, i + c0 - off[i]))
        else:
            d = off[i + 1] - c0
            la = i + c0 - off[i]
            lb = (i + 1) + c0 - off[i + 1]
            out.append(("b", j, local, i, la, i + 1, lb, d))
    return out


_BLOCKS = _block_list()


def _emit_row(x_s, q, ba, bb, masks):
    vs = [x_s[q, pl.ds(PADL + L * m, L)] for m in range(D // L)]
    splats = {}

    def splat(i):
        if i not in splats:
            splats[i] = vs[i // L][i % L]
        return splats[i]

    bufs = (ba, bb)
    n = len(_BLOCKS)
    loads = [None] * n
    prods = [None] * n
    for t in range(n + _ST_SKEW):
        if t < n:
            blk = _BLOCKS[t]
            if blk[0] == "s":
                loads[t] = (x_s[q, pl.ds(PADL + blk[4], L)],)
            else:
                loads[t] = (
                    x_s[q, pl.ds(PADL + blk[4], L)],
                    x_s[q, pl.ds(PADL + blk[6], L)],
                )
        tm = t - _MUL_SKEW
        if 0 <= tm < n:
            blk = _BLOCKS[tm]
            if blk[0] == "s":
                prods[tm] = splat(blk[3]) * loads[tm][0]
            else:
                va = splat(blk[3]) * loads[tm][0]
                vb = splat(blk[5]) * loads[tm][1]
                prods[tm] = jnp.where(masks[blk[7]], va, vb)
        ts = t - _ST_SKEW
        if 0 <= ts < n:
            blk = _BLOCKS[ts]
            bufs[blk[1]][q, pl.ds(blk[2], L)] = prods[ts]


def _sc_body(x_hbm, out_hbm, x_s, a0, b0, a1, b1, sa0, sb0, sa1, sb1):
    cid = lax.axis_index("c")
    sid = lax.axis_index("s")
    wid = sid * NC + cid
    base = wid * RPW
    sets = ((a0, b0, sa0, sb0), (a1, b1, sa1, sb1))
    iota = lax.broadcasted_iota(jnp.int32, (L,), 0)
    masks = {d: iota < d for d in range(1, L)}

    def stripe(t, s, ba, bb, sema, semb):
        row0 = base + s * SR
        pltpu.sync_copy(x_hbm.at[pl.ds(row0, SR)], x_s)

        @pl.when(t > 0)
        def _wait():
            pltpu.make_async_copy(
                ba, out_hbm.at[pl.ds(base, SR), pl.ds(0, HALF)], sema
            ).wait()
            pltpu.make_async_copy(
                bb, out_hbm.at[pl.ds(base, SR), pl.ds(HALF, HB)], semb
            ).wait()

        def row(q, carry):
            _emit_row(x_s, q, ba, bb, masks)
            return carry

        lax.fori_loop(0, SR, row, 0)
        pltpu.async_copy(ba, out_hbm.at[pl.ds(row0, SR), pl.ds(0, HALF)], sema)
        pltpu.async_copy(bb, out_hbm.at[pl.ds(row0, SR), pl.ds(HALF, HB)], semb)

    def pair(t, carry):
        for k, (ba, bb, sema, semb) in enumerate(sets):
            stripe(t, t * 2 + k, ba, bb, sema, semb)
        return carry

    lax.fori_loop(0, NSTR // 2, pair, 0)
    for ba, bb, sema, semb in sets:
        pltpu.make_async_copy(
            ba, out_hbm.at[pl.ds(base, SR), pl.ds(0, HALF)], sema
        ).wait()
        pltpu.make_async_copy(
            bb, out_hbm.at[pl.ds(base, SR), pl.ds(HALF, HB)], semb
        ).wait()


_mesh = plsc.VectorSubcoreMesh(core_axis_name="c", subcore_axis_name="s")

_sc_call = pl.kernel(
    _sc_body,
    out_type=jax.ShapeDtypeStruct((B, C), jnp.float32),
    mesh=_mesh,
    scratch_types=[
        pltpu.VMEM((SR, PADW), jnp.float32),
        pltpu.VMEM((SR, HALF), jnp.float32),
        pltpu.VMEM((SR, HB), jnp.float32),
        pltpu.VMEM((SR, HALF), jnp.float32),
        pltpu.VMEM((SR, HB), jnp.float32),
        pltpu.SemaphoreType.DMA,
        pltpu.SemaphoreType.DMA,
        pltpu.SemaphoreType.DMA,
        pltpu.SemaphoreType.DMA,
    ],
)


def _sel_matrices():
    pairs = []
    off = 0
    for i in range(D):
        for j in range(i, D):
            pairs.append((i, j))
    si = np.zeros((D, 256), np.float32)
    sj = np.zeros((D, 256), np.float32)
    for cc in range(256):
        c = SC_END + cc
        if c < C:
            i, j = pairs[c]
            si[i, cc] = 1.0
            sj[j, cc] = 1.0
    return si, sj


_SEL_I, _SEL_J = _sel_matrices()


def _tc_body(x_ref, si_ref, sj_ref, prev_ref, o_ref):
    xv = x_ref[...]
    a = jnp.dot(xv, si_ref[...], preferred_element_type=jnp.float32,
                precision=lax.Precision.HIGHEST)
    b = jnp.dot(xv, sj_ref[...], preferred_element_type=jnp.float32,
                precision=lax.Precision.HIGHEST)
    o_ref[...] = a * b


_tc_call = pl.pallas_call(
    _tc_body,
    grid=(8, 2),
    in_specs=[
        pl.BlockSpec((512, D), lambda b, j: (b, 0)),
        pl.BlockSpec((D, 128), lambda b, j: (0, j)),
        pl.BlockSpec((D, 128), lambda b, j: (0, j)),
        pl.BlockSpec((512, 128), lambda b, j: (b, 63 + j)),
    ],
    out_specs=pl.BlockSpec((512, 128), lambda b, j: (b, 63 + j)),
    out_shape=jax.ShapeDtypeStruct((B, C), jnp.float32),
    input_output_aliases={3: 0},
)


@jax.jit
def kernel(x):
    assert x.shape == (B, D) and x.dtype == jnp.float32
    xp = jnp.pad(x, ((0, 0), (PADL, PADW - PADL - D)))
    part = _sc_call(xp)
    return _tc_call(x, _SEL_I, _SEL_J, part)
